# Initial kernel scaffold; baseline (speedup 1.0000x reference)
import jax
import jax.numpy as jnp
from jax.experimental import pallas as pl


def kernel(features, memory_bank):
    raise NotImplementedError("write your pallas kernel here")


# speedup vs baseline: 3.3583x; 3.3583x over previous
import functools

import jax
import jax.numpy as jnp
from jax.experimental import pallas as pl
from jax.experimental.pallas import tpu as pltpu


def _knn_body(B, C, HW, BK, nsteps, qf_ref, mb_ref, out_ref,
              qn_ref, qsq_ref, acc_ref):
    Q = B * HW
    j = pl.program_id(0)

    @pl.when(j == 0)
    def _init():
        for b in range(B):
            f = qf_ref[b * C:(b + 1) * C, :]
            nrm = jnp.sqrt(jnp.sum(f * f, axis=0, keepdims=True))
            qn = f / (nrm + 1e-12)
            qn_ref[:, b * HW:(b + 1) * HW] = qn.astype(jnp.bfloat16)
            qsq_ref[0:1, b * HW:(b + 1) * HW] = jnp.sum(
                qn * qn, axis=0, keepdims=True)
        acc_ref[...] = jnp.full((1, Q), jnp.inf, dtype=jnp.float32)

    mb = mb_ref[...]
    ksq = jnp.sum(mb * mb, axis=1, keepdims=True)
    mbn = mb / (jnp.sqrt(ksq) + 1e-12)
    ksqn = jnp.sum(mbn * mbn, axis=1, keepdims=True)
    s = jax.lax.dot_general(
        mbn.astype(jnp.bfloat16), qn_ref[...],
        (((1,), (0,)), ((), ())),
        preferred_element_type=jnp.float32)
    part = ksqn - 2.0 * s
    acc_ref[...] = jnp.minimum(acc_ref[...],
                               jnp.min(part, axis=0, keepdims=True))

    @pl.when(j == nsteps - 1)
    def _finish():
        d2 = acc_ref[...] + qsq_ref[...]
        d2b = jnp.broadcast_to(d2, (B, Q))
        col = jax.lax.broadcasted_iota(jnp.int32, (B, Q), 1)
        row = jax.lax.broadcasted_iota(jnp.int32, (B, Q), 0)
        masked = jnp.where(col // HW == row, d2b, -jnp.inf)
        out_ref[...] = jnp.max(masked, axis=1, keepdims=True)


def kernel(features, memory_bank):
    B, C, H, W = features.shape
    K, C2 = memory_bank.shape
    HW = H * W
    Q = B * HW
    BK = 1000
    nsteps = K // BK
    qf = features.reshape(B * C, HW)

    out = pl.pallas_call(
        functools.partial(_knn_body, B, C, HW, BK, nsteps),
        grid=(nsteps,),
        in_specs=[
            pl.BlockSpec((B * C, HW), lambda j: (0, 0)),
            pl.BlockSpec((BK, C), lambda j: (j, 0)),
        ],
        out_specs=pl.BlockSpec((B, 1), lambda j: (0, 0)),
        out_shape=jax.ShapeDtypeStruct((B, 1), jnp.float32),
        scratch_shapes=[
            pltpu.VMEM((C, Q), jnp.bfloat16),
            pltpu.VMEM((1, Q), jnp.float32),
            pltpu.VMEM((1, Q), jnp.float32),
        ],
        compiler_params=pltpu.CompilerParams(
            dimension_semantics=("arbitrary",)),
    )(qf, memory_bank)
    return out.reshape(B)
